# baseline (speedup 1.0000x reference)
import functools

import numpy as np
import jax
import jax.numpy as jnp
from jax.experimental import pallas as pl
from jax.experimental.pallas import tpu as pltpu
from jax.sharding import Mesh, PartitionSpec as P

try:
    from jax import shard_map as _shard_map

    def _smap(f, mesh, in_specs, out_specs):
        return _shard_map(f, mesh=mesh, in_specs=in_specs,
                          out_specs=out_specs, check_vma=False)
except ImportError:
    from jax.experimental.shard_map import shard_map as _shard_map

    def _smap(f, mesh, in_specs, out_specs):
        return _shard_map(f, mesh=mesh, in_specs=in_specs,
                          out_specs=out_specs, check_rep=False)

_TINY = np.float32(np.finfo(np.float32).tiny)
_BLK = 8192
_SLC = 256

_K0 = np.uint32(0)
_K1 = np.uint32(42)
_KS2 = np.uint32(np.uint32(0x1BD11BDA) ^ _K0 ^ _K1)
_ROT0 = (13, 15, 26, 6)
_ROT1 = (17, 29, 16, 24)


def _rotl(x, d):
    return (x << np.uint32(d)) | (x >> np.uint32(32 - d))


def _threefry_xor(x1):
    def rounds(x0, x1, rots):
        for r in rots:
            x0 = x0 + x1
            x1 = x0 ^ _rotl(x1, r)
        return x0, x1

    x0 = x1
    x1 = x0 ^ _rotl(x1, _ROT0[0])
    x0, x1 = rounds(x0, x1, _ROT0[1:])
    x0 = x0 + _K1
    x1 = x1 + np.uint32(_KS2 + np.uint32(1))
    x0, x1 = rounds(x0, x1, _ROT1)
    x0 = x0 + _KS2
    x1 = x1 + np.uint32(_K0 + np.uint32(2))
    x0, x1 = rounds(x0, x1, _ROT0)
    x0 = x0 + _K0
    x1 = x1 + np.uint32(_K1 + np.uint32(3))
    x0, x1 = rounds(x0, x1, _ROT1)
    x0 = x0 + _K1
    x1 = x1 + np.uint32(_KS2 + np.uint32(4))
    x0, x1 = rounds(x0, x1, _ROT0)
    x0 = x0 + _KS2
    x1 = x1 + np.uint32(_K0 + np.uint32(5))
    return x0 ^ x1


def _gumbel(bits):
    fb = (bits >> np.uint32(9)) | np.uint32(0x3F800000)
    f = jax.lax.bitcast_convert_type(fb, jnp.float32) - np.float32(1.0)
    u = jnp.maximum(_TINY, f * np.float32(np.float32(1.0) - _TINY) + _TINY)
    return -jnp.log(-jnp.log(u))


def _sampler_kernel(off_ref, logits_ref, val_ref, idx_ref, vmax_ref, vidx_ref,
                    *, nblk, Vs, V, B):
    k = pl.program_id(0)
    off = off_ref[0]

    @pl.when(k == 0)
    def _init():
        vmax_ref[...] = jnp.full((B, _SLC), -jnp.inf, jnp.float32)
        vidx_ref[...] = jnp.zeros((B, _SLC), jnp.int32)

    nslc = _BLK // _SLC
    base_col = k * _BLK
    rv = (jax.lax.broadcasted_iota(jnp.uint32, (B, _SLC), 0) * np.uint32(V)
          + jax.lax.broadcasted_iota(jnp.uint32, (B, _SLC), 1))
    coli = jax.lax.broadcasted_iota(jnp.int32, (B, _SLC), 1)

    def body(s, carry):
        vmax, vidx = carry
        col0 = s * _SLC
        x1 = rv + (off + base_col + col0 + 42).astype(jnp.uint32)
        g = _gumbel(_threefry_xor(x1))
        vals = logits_ref[:, pl.ds(col0, _SLC)] + g
        vals = jnp.where(coli < Vs - (base_col + col0), vals, -jnp.inf)
        upd = vals > vmax
        sid = jnp.full((B, _SLC), 0, jnp.int32) + (k * nslc + s)
        return jnp.where(upd, vals, vmax), jnp.where(upd, sid, vidx)

    vmax, vidx = jax.lax.fori_loop(
        0, nslc, body, (vmax_ref[...], vidx_ref[...]))
    vmax_ref[...] = vmax
    vidx_ref[...] = vidx

    @pl.when(k == nblk - 1)
    def _done():
        vm = vmax_ref[...]
        m = jnp.max(vm, axis=1, keepdims=True)
        gidx = vidx_ref[...] * _SLC + coli + off
        val_ref[...] = m
        idx_ref[...] = jnp.min(
            jnp.where(vm == m, gidx, np.int32(2**30)), axis=1, keepdims=True)


def _local_top1(logits_shard, off, V):
    B, Vs = logits_shard.shape
    nblk = (Vs + _BLK - 1) // _BLK
    off_arr = jnp.reshape(off, (1,)).astype(jnp.int32)
    return pl.pallas_call(
        functools.partial(_sampler_kernel, nblk=nblk, Vs=Vs, V=V, B=B),
        grid=(nblk,),
        in_specs=[pl.BlockSpec(memory_space=pltpu.SMEM),
                  pl.BlockSpec((B, _BLK), lambda k: (0, k))],
        out_specs=[pl.BlockSpec((B, 1), lambda k: (0, 0)),
                   pl.BlockSpec((B, 1), lambda k: (0, 0))],
        out_shape=[jax.ShapeDtypeStruct((B, 1), jnp.float32),
                   jax.ShapeDtypeStruct((B, 1), jnp.int32)],
        scratch_shapes=[pltpu.VMEM((B, _SLC), jnp.float32),
                        pltpu.VMEM((B, _SLC), jnp.int32)],
    )(off_arr, logits_shard)


def kernel(logits):
    B, V = logits.shape
    devs = jax.devices()
    n = len(devs)
    while n > 1 and V % n:
        n -= 1
    if n <= 1:
        _, idx = _local_top1(logits, jnp.int32(0), V)
        return idx

    Vs = V // n
    mesh = Mesh(np.array(devs[:n]), ("x",))

    def shard_fn(lg):
        r = jax.lax.axis_index("x")
        v, i = _local_top1(lg, (r * Vs).astype(jnp.int32), V)
        vg = jax.lax.all_gather(v[:, 0], "x")
        ig = jax.lax.all_gather(i[:, 0], "x")
        best = jnp.argmax(vg, axis=0)
        return jnp.take_along_axis(ig, best[None, :], axis=0).T

    f = _smap(shard_fn, mesh, P(None, "x"), P())
    return f(logits)


# speedup vs baseline: 1.5690x; 1.5690x over previous
import functools

import numpy as np
import jax
import jax.numpy as jnp
from jax.experimental import pallas as pl
from jax.experimental.pallas import tpu as pltpu
from jax.experimental.pallas import tpu_sc as plsc

_TINY = np.float32(np.finfo(np.float32).tiny)
_BLK = 8192
_SLC = 256
_SC_BLOCKS = 25
_SC_CHUNK = 8192

_K0 = np.uint32(0)
_K1 = np.uint32(42)
_KS2 = np.uint32(np.uint32(0x1BD11BDA) ^ _K0 ^ _K1)
_ROT0 = (13, 15, 26, 6)
_ROT1 = (17, 29, 16, 24)


def _rotl(x, d):
    return (x << np.uint32(d)) | (x >> np.uint32(32 - d))


def _threefry_xor(x1):
    def rounds(x0, x1, rots):
        for r in rots:
            x0 = x0 + x1
            x1 = x0 ^ _rotl(x1, r)
        return x0, x1

    x0 = x1
    x1 = x0 ^ _rotl(x1, _ROT0[0])
    x0, x1 = rounds(x0, x1, _ROT0[1:])
    x0 = x0 + _K1
    x1 = x1 + np.uint32(_KS2 + np.uint32(1))
    x0, x1 = rounds(x0, x1, _ROT1)
    x0 = x0 + _KS2
    x1 = x1 + np.uint32(_K0 + np.uint32(2))
    x0, x1 = rounds(x0, x1, _ROT0)
    x0 = x0 + _K0
    x1 = x1 + np.uint32(_K1 + np.uint32(3))
    x0, x1 = rounds(x0, x1, _ROT1)
    x0 = x0 + _K1
    x1 = x1 + np.uint32(_KS2 + np.uint32(4))
    x0, x1 = rounds(x0, x1, _ROT0)
    x0 = x0 + _KS2
    x1 = x1 + np.uint32(_K0 + np.uint32(5))
    return x0 ^ x1


def _bits_to_u(bits):
    fb = (bits >> np.uint32(9)) | np.uint32(0x3F800000)
    f = jax.lax.bitcast_convert_type(fb, jnp.float32) - np.float32(1.0)
    return jnp.maximum(_TINY, f * np.float32(np.float32(1.0) - _TINY) + _TINY)


def _sc_body(u_ref, buf, *, V, col0, n_cols):
    b = jax.lax.axis_index("s") * 2 + jax.lax.axis_index("c")
    base = b * V + col0 + 42
    lane = jax.lax.iota(jnp.int32, 16)
    nch = n_cols // _SC_CHUNK

    def chunk(ci, _):
        def vec(vi, _):
            off = vi * 64
            for t in range(4):
                x1 = (lane + (base + ci * _SC_CHUNK + off + t * 16)
                      ).astype(jnp.uint32)
                buf[pl.ds(off + t * 16, 16)] = _bits_to_u(_threefry_xor(x1))
            return 0

        jax.lax.fori_loop(0, _SC_CHUNK // 64, vec, 0)
        pltpu.sync_copy(buf, u_ref.at[b, pl.ds(ci * _SC_CHUNK, _SC_CHUNK)])
        return 0

    jax.lax.fori_loop(0, nch, chunk, 0)


def _sc_u_values(B, V, col0, n_cols):
    mesh = plsc.VectorSubcoreMesh(core_axis_name="c", subcore_axis_name="s")
    body = functools.partial(_sc_body, V=V, col0=col0, n_cols=n_cols)
    try:
        fn = pl.kernel(
            body, mesh=mesh,
            out_type=jax.ShapeDtypeStruct((B, n_cols), jnp.float32),
            scratch_types=[pltpu.VMEM((_SC_CHUNK,), jnp.float32)],
        )
    except TypeError:
        fn = pl.kernel(
            body, mesh=mesh,
            out_shape=jax.ShapeDtypeStruct((B, n_cols), jnp.float32),
            scratch_shapes=[pltpu.VMEM((_SC_CHUNK,), jnp.float32)],
        )
    return fn()


def _tc1_kernel(logits_ref, vmax_ref, vidx_ref, *, V, B):
    k = pl.program_id(0)

    @pl.when(k == 0)
    def _init():
        vmax_ref[...] = jnp.full((B, _SLC), -jnp.inf, jnp.float32)
        vidx_ref[...] = jnp.zeros((B, _SLC), jnp.int32)

    nslc = _BLK // _SLC
    base_col = k * _BLK
    rv = (jax.lax.broadcasted_iota(jnp.uint32, (B, _SLC), 0) * np.uint32(V)
          + jax.lax.broadcasted_iota(jnp.uint32, (B, _SLC), 1))

    def body(s, carry):
        vmax, vidx = carry
        col0 = s * _SLC
        x1 = rv + (base_col + col0 + 42).astype(jnp.uint32)
        u = _bits_to_u(_threefry_xor(x1))
        g = -jnp.log(-jnp.log(u))
        vals = logits_ref[:, pl.ds(col0, _SLC)] + g
        upd = vals > vmax
        sid = jnp.full((B, _SLC), 0, jnp.int32) + (k * nslc + s)
        return jnp.where(upd, vals, vmax), jnp.where(upd, sid, vidx)

    vmax, vidx = jax.lax.fori_loop(
        0, nslc, body, (vmax_ref[...], vidx_ref[...]))
    vmax_ref[...] = vmax
    vidx_ref[...] = vidx


def _tc2_kernel(logits_ref, u_ref, pv_ref, pi_ref, out_ref, vmax_ref, vidx_ref,
                *, nblk2, k_off, V, B):
    k = pl.program_id(0)

    @pl.when(k == 0)
    def _init():
        vmax_ref[...] = pv_ref[...]
        vidx_ref[...] = pi_ref[...]

    nslc = _BLK // _SLC
    kg = k + k_off
    base_col = kg * _BLK
    coli = jax.lax.broadcasted_iota(jnp.int32, (B, _SLC), 1)

    def body(s, carry):
        vmax, vidx = carry
        col0 = s * _SLC
        g = -jnp.log(-jnp.log(u_ref[:, pl.ds(col0, _SLC)]))
        vals = logits_ref[:, pl.ds(col0, _SLC)] + g
        vals = jnp.where(coli < V - (base_col + col0), vals, -jnp.inf)
        upd = vals > vmax
        sid = jnp.full((B, _SLC), 0, jnp.int32) + (kg * nslc + s)
        return jnp.where(upd, vals, vmax), jnp.where(upd, sid, vidx)

    vmax, vidx = jax.lax.fori_loop(
        0, nslc, body, (vmax_ref[...], vidx_ref[...]))
    vmax_ref[...] = vmax
    vidx_ref[...] = vidx

    @pl.when(k == nblk2 - 1)
    def _done():
        vm = vmax_ref[...]
        m = jnp.max(vm, axis=1, keepdims=True)
        gidx = vidx_ref[...] * _SLC + coli
        out_ref[...] = jnp.min(
            jnp.where(vm == m, gidx, np.int32(2**30)), axis=1, keepdims=True)


def kernel(logits):
    B, V = logits.shape
    nblk = (V + _BLK - 1) // _BLK
    nblk2 = min(_SC_BLOCKS, nblk - 1)
    nblk1 = nblk - nblk2

    u_tail = _sc_u_values(B, V, nblk1 * _BLK, nblk2 * _BLK)

    pv, pi = pl.pallas_call(
        functools.partial(_tc1_kernel, V=V, B=B),
        grid=(nblk1,),
        in_specs=[pl.BlockSpec((B, _BLK), lambda k: (0, k))],
        out_specs=[pl.BlockSpec((B, _SLC), lambda k: (0, 0)),
                   pl.BlockSpec((B, _SLC), lambda k: (0, 0))],
        out_shape=[jax.ShapeDtypeStruct((B, _SLC), jnp.float32),
                   jax.ShapeDtypeStruct((B, _SLC), jnp.int32)],
    )(logits)

    return pl.pallas_call(
        functools.partial(_tc2_kernel, nblk2=nblk2, k_off=nblk1, V=V, B=B),
        grid=(nblk2,),
        in_specs=[
            pl.BlockSpec((B, _BLK), lambda k, o=nblk1: (0, k + o)),
            pl.BlockSpec((B, _BLK), lambda k: (0, k)),
            pl.BlockSpec((B, _SLC), lambda k: (0, 0)),
            pl.BlockSpec((B, _SLC), lambda k: (0, 0)),
        ],
        out_specs=pl.BlockSpec((B, 1), lambda k: (0, 0)),
        out_shape=jax.ShapeDtypeStruct((B, 1), jnp.int32),
        scratch_shapes=[pltpu.VMEM((B, _SLC), jnp.float32),
                        pltpu.VMEM((B, _SLC), jnp.int32)],
    )(logits, u_tail, pv, pi)


# speedup vs baseline: 1.6141x; 1.0288x over previous
import functools

import numpy as np
import jax
import jax.numpy as jnp
from jax.experimental import pallas as pl
from jax.experimental.pallas import tpu as pltpu
from jax.experimental.pallas import tpu_sc as plsc

_TINY = np.float32(np.finfo(np.float32).tiny)
_BLK = 8192
_SLC = 256
_SC_BLOCKS = 37
_SC_CHUNK = 8192

_K0 = np.uint32(0)
_K1 = np.uint32(42)
_KS2 = np.uint32(np.uint32(0x1BD11BDA) ^ _K0 ^ _K1)
_ROT0 = (13, 15, 26, 6)
_ROT1 = (17, 29, 16, 24)


def _rotl(x, d):
    return (x << np.uint32(d)) | (x >> np.uint32(32 - d))


def _threefry_xor(x1):
    def rounds(x0, x1, rots):
        for r in rots:
            x0 = x0 + x1
            x1 = x0 ^ _rotl(x1, r)
        return x0, x1

    x0 = x1
    x1 = x0 ^ _rotl(x1, _ROT0[0])
    x0, x1 = rounds(x0, x1, _ROT0[1:])
    x0 = x0 + _K1
    x1 = x1 + np.uint32(_KS2 + np.uint32(1))
    x0, x1 = rounds(x0, x1, _ROT1)
    x0 = x0 + _KS2
    x1 = x1 + np.uint32(_K0 + np.uint32(2))
    x0, x1 = rounds(x0, x1, _ROT0)
    x0 = x0 + _K0
    x1 = x1 + np.uint32(_K1 + np.uint32(3))
    x0, x1 = rounds(x0, x1, _ROT1)
    x0 = x0 + _K1
    x1 = x1 + np.uint32(_KS2 + np.uint32(4))
    x0, x1 = rounds(x0, x1, _ROT0)
    x0 = x0 + _KS2
    x1 = x1 + np.uint32(_K0 + np.uint32(5))
    return x0 ^ x1


def _bits_to_u(bits):
    fb = (bits >> np.uint32(9)) | np.uint32(0x3F800000)
    f = jax.lax.bitcast_convert_type(fb, jnp.float32) - np.float32(1.0)
    return jnp.maximum(_TINY, f * np.float32(np.float32(1.0) - _TINY) + _TINY)


def _sc_body(u_ref, buf, *, V, col0, n_cols):
    b = jax.lax.axis_index("s") * 2 + jax.lax.axis_index("c")
    base = b * V + col0 + 42
    lane = jax.lax.iota(jnp.int32, 16)
    nch = n_cols // _SC_CHUNK

    def chunk(ci, _):
        def vec(vi, _):
            off = vi * 64
            for t in range(4):
                x1 = (lane + (base + ci * _SC_CHUNK + off + t * 16)
                      ).astype(jnp.uint32)
                buf[pl.ds(off + t * 16, 16)] = _bits_to_u(_threefry_xor(x1))
            return 0

        jax.lax.fori_loop(0, _SC_CHUNK // 64, vec, 0)
        pltpu.sync_copy(buf, u_ref.at[b, pl.ds(ci * _SC_CHUNK, _SC_CHUNK)])
        return 0

    jax.lax.fori_loop(0, nch, chunk, 0)


def _sc_u_values(B, V, col0, n_cols):
    mesh = plsc.VectorSubcoreMesh(core_axis_name="c", subcore_axis_name="s")
    body = functools.partial(_sc_body, V=V, col0=col0, n_cols=n_cols)
    try:
        fn = pl.kernel(
            body, mesh=mesh,
            out_type=jax.ShapeDtypeStruct((B, n_cols), jnp.float32),
            scratch_types=[pltpu.VMEM((_SC_CHUNK,), jnp.float32)],
        )
    except TypeError:
        fn = pl.kernel(
            body, mesh=mesh,
            out_shape=jax.ShapeDtypeStruct((B, n_cols), jnp.float32),
            scratch_shapes=[pltpu.VMEM((_SC_CHUNK,), jnp.float32)],
        )
    return fn()


def _tc1_kernel(logits_ref, vmax_ref, vidx_ref, *, V, B):
    k = pl.program_id(0)

    @pl.when(k == 0)
    def _init():
        vmax_ref[...] = jnp.full((B, _SLC), -jnp.inf, jnp.float32)
        vidx_ref[...] = jnp.zeros((B, _SLC), jnp.int32)

    nslc = _BLK // _SLC
    base_col = k * _BLK
    rv = (jax.lax.broadcasted_iota(jnp.uint32, (B, _SLC), 0) * np.uint32(V)
          + jax.lax.broadcasted_iota(jnp.uint32, (B, _SLC), 1))

    def body(s, carry):
        vmax, vidx = carry
        col0 = s * _SLC
        x1 = rv + (base_col + col0 + 42).astype(jnp.uint32)
        u = _bits_to_u(_threefry_xor(x1))
        g = -jnp.log(-jnp.log(u))
        vals = logits_ref[:, pl.ds(col0, _SLC)] + g
        upd = vals > vmax
        sid = jnp.full((B, _SLC), 0, jnp.int32) + (k * nslc + s)
        return jnp.where(upd, vals, vmax), jnp.where(upd, sid, vidx)

    vmax, vidx = jax.lax.fori_loop(
        0, nslc, body, (vmax_ref[...], vidx_ref[...]))
    vmax_ref[...] = vmax
    vidx_ref[...] = vidx


def _tc2_kernel(logits_ref, u_ref, pv_ref, pi_ref, out_ref, vmax_ref, vidx_ref,
                *, nblk2, k_off, V, B):
    k = pl.program_id(0)

    @pl.when(k == 0)
    def _init():
        vmax_ref[...] = pv_ref[...]
        vidx_ref[...] = pi_ref[...]

    nslc = _BLK // _SLC
    kg = k + k_off
    base_col = kg * _BLK
    coli = jax.lax.broadcasted_iota(jnp.int32, (B, _SLC), 1)

    def body(s, carry):
        vmax, vidx = carry
        col0 = s * _SLC
        g = -jnp.log(-jnp.log(u_ref[:, pl.ds(col0, _SLC)]))
        vals = logits_ref[:, pl.ds(col0, _SLC)] + g
        vals = jnp.where(coli < V - (base_col + col0), vals, -jnp.inf)
        upd = vals > vmax
        sid = jnp.full((B, _SLC), 0, jnp.int32) + (kg * nslc + s)
        return jnp.where(upd, vals, vmax), jnp.where(upd, sid, vidx)

    vmax, vidx = jax.lax.fori_loop(
        0, nslc, body, (vmax_ref[...], vidx_ref[...]))
    vmax_ref[...] = vmax
    vidx_ref[...] = vidx

    @pl.when(k == nblk2 - 1)
    def _done():
        vm = vmax_ref[...]
        m = jnp.max(vm, axis=1, keepdims=True)
        gidx = vidx_ref[...] * _SLC + coli
        out_ref[...] = jnp.min(
            jnp.where(vm == m, gidx, np.int32(2**30)), axis=1, keepdims=True)


def kernel(logits):
    B, V = logits.shape
    nblk = (V + _BLK - 1) // _BLK
    nblk2 = min(_SC_BLOCKS, nblk - 1)
    nblk1 = nblk - nblk2

    u_tail = _sc_u_values(B, V, nblk1 * _BLK, nblk2 * _BLK)

    pv, pi = pl.pallas_call(
        functools.partial(_tc1_kernel, V=V, B=B),
        grid=(nblk1,),
        in_specs=[pl.BlockSpec((B, _BLK), lambda k: (0, k))],
        out_specs=[pl.BlockSpec((B, _SLC), lambda k: (0, 0)),
                   pl.BlockSpec((B, _SLC), lambda k: (0, 0))],
        out_shape=[jax.ShapeDtypeStruct((B, _SLC), jnp.float32),
                   jax.ShapeDtypeStruct((B, _SLC), jnp.int32)],
    )(logits)

    return pl.pallas_call(
        functools.partial(_tc2_kernel, nblk2=nblk2, k_off=nblk1, V=V, B=B),
        grid=(nblk2,),
        in_specs=[
            pl.BlockSpec((B, _BLK), lambda k, o=nblk1: (0, k + o)),
            pl.BlockSpec((B, _BLK), lambda k: (0, k)),
            pl.BlockSpec((B, _SLC), lambda k: (0, 0)),
            pl.BlockSpec((B, _SLC), lambda k: (0, 0)),
        ],
        out_specs=pl.BlockSpec((B, 1), lambda k: (0, 0)),
        out_shape=jax.ShapeDtypeStruct((B, 1), jnp.int32),
        scratch_shapes=[pltpu.VMEM((B, _SLC), jnp.float32),
                        pltpu.VMEM((B, _SLC), jnp.int32)],
    )(logits, u_tail, pv, pi)


# speedup vs baseline: 1.6220x; 1.0049x over previous
import functools

import numpy as np
import jax
import jax.numpy as jnp
from jax.experimental import pallas as pl
from jax.experimental.pallas import tpu as pltpu
from jax.experimental.pallas import tpu_sc as plsc

_TINY = np.float32(np.finfo(np.float32).tiny)
_BLK = 8192
_SLC = 512
_SC_BLOCKS = 37
_SC_CHUNK = 8192

_K0 = np.uint32(0)
_K1 = np.uint32(42)
_KS2 = np.uint32(np.uint32(0x1BD11BDA) ^ _K0 ^ _K1)
_ROT0 = (13, 15, 26, 6)
_ROT1 = (17, 29, 16, 24)


def _rotl(x, d):
    return (x << np.uint32(d)) | (x >> np.uint32(32 - d))


def _threefry_xor(x1):
    def rounds(x0, x1, rots):
        for r in rots:
            x0 = x0 + x1
            x1 = x0 ^ _rotl(x1, r)
        return x0, x1

    x0 = x1
    x1 = x0 ^ _rotl(x1, _ROT0[0])
    x0, x1 = rounds(x0, x1, _ROT0[1:])
    x0 = x0 + _K1
    x1 = x1 + np.uint32(_KS2 + np.uint32(1))
    x0, x1 = rounds(x0, x1, _ROT1)
    x0 = x0 + _KS2
    x1 = x1 + np.uint32(_K0 + np.uint32(2))
    x0, x1 = rounds(x0, x1, _ROT0)
    x0 = x0 + _K0
    x1 = x1 + np.uint32(_K1 + np.uint32(3))
    x0, x1 = rounds(x0, x1, _ROT1)
    x0 = x0 + _K1
    x1 = x1 + np.uint32(_KS2 + np.uint32(4))
    x0, x1 = rounds(x0, x1, _ROT0)
    x0 = x0 + _KS2
    x1 = x1 + np.uint32(_K0 + np.uint32(5))
    return x0 ^ x1


def _bits_to_u(bits):
    fb = (bits >> np.uint32(9)) | np.uint32(0x3F800000)
    f = jax.lax.bitcast_convert_type(fb, jnp.float32) - np.float32(1.0)
    return jnp.maximum(_TINY, f * np.float32(np.float32(1.0) - _TINY) + _TINY)


def _sc_body(u_ref, buf, *, V, col0, n_cols):
    b = jax.lax.axis_index("s") * 2 + jax.lax.axis_index("c")
    base = b * V + col0 + 42
    lane = jax.lax.iota(jnp.int32, 16)
    nch = n_cols // _SC_CHUNK

    def chunk(ci, _):
        def vec(vi, _):
            off = vi * 64
            for t in range(4):
                x1 = (lane + (base + ci * _SC_CHUNK + off + t * 16)
                      ).astype(jnp.uint32)
                buf[pl.ds(off + t * 16, 16)] = _bits_to_u(_threefry_xor(x1))
            return 0

        jax.lax.fori_loop(0, _SC_CHUNK // 64, vec, 0)
        pltpu.sync_copy(buf, u_ref.at[b, pl.ds(ci * _SC_CHUNK, _SC_CHUNK)])
        return 0

    jax.lax.fori_loop(0, nch, chunk, 0)


def _sc_u_values(B, V, col0, n_cols):
    mesh = plsc.VectorSubcoreMesh(core_axis_name="c", subcore_axis_name="s")
    body = functools.partial(_sc_body, V=V, col0=col0, n_cols=n_cols)
    try:
        fn = pl.kernel(
            body, mesh=mesh,
            out_type=jax.ShapeDtypeStruct((B, n_cols), jnp.float32),
            scratch_types=[pltpu.VMEM((_SC_CHUNK,), jnp.float32)],
        )
    except TypeError:
        fn = pl.kernel(
            body, mesh=mesh,
            out_shape=jax.ShapeDtypeStruct((B, n_cols), jnp.float32),
            scratch_shapes=[pltpu.VMEM((_SC_CHUNK,), jnp.float32)],
        )
    return fn()


def _tc1_kernel(logits_ref, vmax_ref, vidx_ref, *, V, B):
    k = pl.program_id(0)

    @pl.when(k == 0)
    def _init():
        vmax_ref[...] = jnp.full((B, _SLC), -jnp.inf, jnp.float32)
        vidx_ref[...] = jnp.zeros((B, _SLC), jnp.int32)

    nslc = _BLK // _SLC
    base_col = k * _BLK
    rv = (jax.lax.broadcasted_iota(jnp.uint32, (B, _SLC), 0) * np.uint32(V)
          + jax.lax.broadcasted_iota(jnp.uint32, (B, _SLC), 1))

    def body(s, carry):
        vmax, vidx = carry
        col0 = s * _SLC
        x1 = rv + (base_col + col0 + 42).astype(jnp.uint32)
        u = _bits_to_u(_threefry_xor(x1))
        g = -jnp.log(-jnp.log(u))
        vals = logits_ref[:, pl.ds(col0, _SLC)] + g
        upd = vals > vmax
        sid = jnp.full((B, _SLC), 0, jnp.int32) + (k * nslc + s)
        return jnp.where(upd, vals, vmax), jnp.where(upd, sid, vidx)

    vmax, vidx = jax.lax.fori_loop(
        0, nslc, body, (vmax_ref[...], vidx_ref[...]))
    vmax_ref[...] = vmax
    vidx_ref[...] = vidx


def _tc2_kernel(logits_ref, u_ref, pv_ref, pi_ref, out_ref, vmax_ref, vidx_ref,
                *, nblk2, k_off, V, B):
    k = pl.program_id(0)

    @pl.when(k == 0)
    def _init():
        vmax_ref[...] = pv_ref[...]
        vidx_ref[...] = pi_ref[...]

    nslc = _BLK // _SLC
    kg = k + k_off
    base_col = kg * _BLK
    coli = jax.lax.broadcasted_iota(jnp.int32, (B, _SLC), 1)

    def body(s, carry):
        vmax, vidx = carry
        col0 = s * _SLC
        g = -jnp.log(-jnp.log(u_ref[:, pl.ds(col0, _SLC)]))
        vals = logits_ref[:, pl.ds(col0, _SLC)] + g
        vals = jnp.where(coli < V - (base_col + col0), vals, -jnp.inf)
        upd = vals > vmax
        sid = jnp.full((B, _SLC), 0, jnp.int32) + (kg * nslc + s)
        return jnp.where(upd, vals, vmax), jnp.where(upd, sid, vidx)

    vmax, vidx = jax.lax.fori_loop(
        0, nslc, body, (vmax_ref[...], vidx_ref[...]))
    vmax_ref[...] = vmax
    vidx_ref[...] = vidx

    @pl.when(k == nblk2 - 1)
    def _done():
        vm = vmax_ref[...]
        m = jnp.max(vm, axis=1, keepdims=True)
        gidx = vidx_ref[...] * _SLC + coli
        out_ref[...] = jnp.min(
            jnp.where(vm == m, gidx, np.int32(2**30)), axis=1, keepdims=True)


def kernel(logits):
    B, V = logits.shape
    nblk = (V + _BLK - 1) // _BLK
    nblk2 = min(_SC_BLOCKS, nblk - 1)
    nblk1 = nblk - nblk2

    u_tail = _sc_u_values(B, V, nblk1 * _BLK, nblk2 * _BLK)

    pv, pi = pl.pallas_call(
        functools.partial(_tc1_kernel, V=V, B=B),
        grid=(nblk1,),
        in_specs=[pl.BlockSpec((B, _BLK), lambda k: (0, k))],
        out_specs=[pl.BlockSpec((B, _SLC), lambda k: (0, 0)),
                   pl.BlockSpec((B, _SLC), lambda k: (0, 0))],
        out_shape=[jax.ShapeDtypeStruct((B, _SLC), jnp.float32),
                   jax.ShapeDtypeStruct((B, _SLC), jnp.int32)],
    )(logits)

    return pl.pallas_call(
        functools.partial(_tc2_kernel, nblk2=nblk2, k_off=nblk1, V=V, B=B),
        grid=(nblk2,),
        in_specs=[
            pl.BlockSpec((B, _BLK), lambda k, o=nblk1: (0, k + o)),
            pl.BlockSpec((B, _BLK), lambda k: (0, k)),
            pl.BlockSpec((B, _SLC), lambda k: (0, 0)),
            pl.BlockSpec((B, _SLC), lambda k: (0, 0)),
        ],
        out_specs=pl.BlockSpec((B, 1), lambda k: (0, 0)),
        out_shape=jax.ShapeDtypeStruct((B, 1), jnp.int32),
        scratch_shapes=[pltpu.VMEM((B, _SLC), jnp.float32),
                        pltpu.VMEM((B, _SLC), jnp.int32)],
    )(logits, u_tail, pv, pi)


# speedup vs baseline: 1.6549x; 1.0203x over previous
import functools

import numpy as np
import jax
import jax.numpy as jnp
from jax.experimental import pallas as pl
from jax.experimental.pallas import tpu as pltpu
from jax.experimental.pallas import tpu_sc as plsc

_TINY = np.float32(np.finfo(np.float32).tiny)
_BLK = 8192
_SLC = 512
_SC_BLOCKS = 32
_SC_CHUNK = 8192

_K0 = np.uint32(0)
_K1 = np.uint32(42)
_KS2 = np.uint32(np.uint32(0x1BD11BDA) ^ _K0 ^ _K1)
_ROT0 = (13, 15, 26, 6)
_ROT1 = (17, 29, 16, 24)


def _rotl(x, d):
    return (x << np.uint32(d)) | (x >> np.uint32(32 - d))


def _threefry_xor(x1):
    def rounds(x0, x1, rots):
        for r in rots:
            x0 = x0 + x1
            x1 = x0 ^ _rotl(x1, r)
        return x0, x1

    x0 = x1
    x1 = x0 ^ _rotl(x1, _ROT0[0])
    x0, x1 = rounds(x0, x1, _ROT0[1:])
    x0 = x0 + _K1
    x1 = x1 + np.uint32(_KS2 + np.uint32(1))
    x0, x1 = rounds(x0, x1, _ROT1)
    x0 = x0 + _KS2
    x1 = x1 + np.uint32(_K0 + np.uint32(2))
    x0, x1 = rounds(x0, x1, _ROT0)
    x0 = x0 + _K0
    x1 = x1 + np.uint32(_K1 + np.uint32(3))
    x0, x1 = rounds(x0, x1, _ROT1)
    x0 = x0 + _K1
    x1 = x1 + np.uint32(_KS2 + np.uint32(4))
    x0, x1 = rounds(x0, x1, _ROT0)
    x0 = x0 + _KS2
    x1 = x1 + np.uint32(_K0 + np.uint32(5))
    return x0 ^ x1


def _bits_to_u(bits):
    fb = (bits >> np.uint32(9)) | np.uint32(0x3F800000)
    f = jax.lax.bitcast_convert_type(fb, jnp.float32) - np.float32(1.0)
    return jnp.maximum(_TINY, f * np.float32(np.float32(1.0) - _TINY) + _TINY)


def _sc_body(u_ref, buf, *, V, col0, n_cols):
    b = jax.lax.axis_index("s") * 2 + jax.lax.axis_index("c")
    base = b * V + col0 + 42
    lane = jax.lax.iota(jnp.int32, 16)
    nch = n_cols // _SC_CHUNK

    def chunk(ci, _):
        def vec(vi, _):
            off = vi * 64
            for t in range(4):
                x1 = (lane + (base + ci * _SC_CHUNK + off + t * 16)
                      ).astype(jnp.uint32)
                buf[pl.ds(off + t * 16, 16)] = _bits_to_u(_threefry_xor(x1))
            return 0

        jax.lax.fori_loop(0, _SC_CHUNK // 64, vec, 0)
        pltpu.sync_copy(buf, u_ref.at[b, pl.ds(ci * _SC_CHUNK, _SC_CHUNK)])
        return 0

    jax.lax.fori_loop(0, nch, chunk, 0)


def _sc_u_values(B, V, col0, n_cols):
    mesh = plsc.VectorSubcoreMesh(core_axis_name="c", subcore_axis_name="s")
    body = functools.partial(_sc_body, V=V, col0=col0, n_cols=n_cols)
    try:
        fn = pl.kernel(
            body, mesh=mesh,
            out_type=jax.ShapeDtypeStruct((B, n_cols), jnp.float32),
            scratch_types=[pltpu.VMEM((_SC_CHUNK,), jnp.float32)],
        )
    except TypeError:
        fn = pl.kernel(
            body, mesh=mesh,
            out_shape=jax.ShapeDtypeStruct((B, n_cols), jnp.float32),
            scratch_shapes=[pltpu.VMEM((_SC_CHUNK,), jnp.float32)],
        )
    return fn()


def _tc1_kernel(logits_ref, vmax_ref, vidx_ref, *, V, B):
    k = pl.program_id(0)

    @pl.when(k == 0)
    def _init():
        vmax_ref[...] = jnp.full((B, _SLC), -jnp.inf, jnp.float32)
        vidx_ref[...] = jnp.zeros((B, _SLC), jnp.int32)

    nslc = _BLK // _SLC
    base_col = k * _BLK
    rv = (jax.lax.broadcasted_iota(jnp.uint32, (B, _SLC), 0) * np.uint32(V)
          + jax.lax.broadcasted_iota(jnp.uint32, (B, _SLC), 1))

    def body(s, carry):
        vmax, vidx = carry
        col0 = s * _SLC
        x1 = rv + (base_col + col0 + 42).astype(jnp.uint32)
        u = _bits_to_u(_threefry_xor(x1))
        g = -jnp.log(-jnp.log(u))
        vals = logits_ref[:, pl.ds(col0, _SLC)] + g
        upd = vals > vmax
        sid = jnp.full((B, _SLC), 0, jnp.int32) + (k * nslc + s)
        return jnp.where(upd, vals, vmax), jnp.where(upd, sid, vidx)

    vmax, vidx = jax.lax.fori_loop(
        0, nslc, body, (vmax_ref[...], vidx_ref[...]))
    vmax_ref[...] = vmax
    vidx_ref[...] = vidx


def _tc2_kernel(logits_ref, u_ref, pv_ref, pi_ref, out_ref, vmax_ref, vidx_ref,
                *, nblk2, k_off, V, B):
    k = pl.program_id(0)

    @pl.when(k == 0)
    def _init():
        vmax_ref[...] = pv_ref[...]
        vidx_ref[...] = pi_ref[...]

    nslc = _BLK // _SLC
    kg = k + k_off
    base_col = kg * _BLK
    coli = jax.lax.broadcasted_iota(jnp.int32, (B, _SLC), 1)

    def body(s, carry):
        vmax, vidx = carry
        col0 = s * _SLC
        g = -jnp.log(-jnp.log(u_ref[:, pl.ds(col0, _SLC)]))
        vals = logits_ref[:, pl.ds(col0, _SLC)] + g
        vals = jnp.where(coli < V - (base_col + col0), vals, -jnp.inf)
        upd = vals > vmax
        sid = jnp.full((B, _SLC), 0, jnp.int32) + (kg * nslc + s)
        return jnp.where(upd, vals, vmax), jnp.where(upd, sid, vidx)

    vmax, vidx = jax.lax.fori_loop(
        0, nslc, body, (vmax_ref[...], vidx_ref[...]))
    vmax_ref[...] = vmax
    vidx_ref[...] = vidx

    @pl.when(k == nblk2 - 1)
    def _done():
        vm = vmax_ref[...]
        m = jnp.max(vm, axis=1, keepdims=True)
        gidx = vidx_ref[...] * _SLC + coli
        out_ref[...] = jnp.min(
            jnp.where(vm == m, gidx, np.int32(2**30)), axis=1, keepdims=True)


def kernel(logits):
    B, V = logits.shape
    nblk = (V + _BLK - 1) // _BLK
    nblk2 = min(_SC_BLOCKS, nblk - 1)
    nblk1 = nblk - nblk2

    u_tail = _sc_u_values(B, V, nblk1 * _BLK, nblk2 * _BLK)

    pv, pi = pl.pallas_call(
        functools.partial(_tc1_kernel, V=V, B=B),
        grid=(nblk1,),
        in_specs=[pl.BlockSpec((B, _BLK), lambda k: (0, k))],
        out_specs=[pl.BlockSpec((B, _SLC), lambda k: (0, 0)),
                   pl.BlockSpec((B, _SLC), lambda k: (0, 0))],
        out_shape=[jax.ShapeDtypeStruct((B, _SLC), jnp.float32),
                   jax.ShapeDtypeStruct((B, _SLC), jnp.int32)],
    )(logits)

    return pl.pallas_call(
        functools.partial(_tc2_kernel, nblk2=nblk2, k_off=nblk1, V=V, B=B),
        grid=(nblk2,),
        in_specs=[
            pl.BlockSpec((B, _BLK), lambda k, o=nblk1: (0, k + o)),
            pl.BlockSpec((B, _BLK), lambda k: (0, k)),
            pl.BlockSpec((B, _SLC), lambda k: (0, 0)),
            pl.BlockSpec((B, _SLC), lambda k: (0, 0)),
        ],
        out_specs=pl.BlockSpec((B, 1), lambda k: (0, 0)),
        out_shape=jax.ShapeDtypeStruct((B, 1), jnp.int32),
        scratch_shapes=[pltpu.VMEM((B, _SLC), jnp.float32),
                        pltpu.VMEM((B, _SLC), jnp.int32)],
    )(logits, u_tail, pv, pi)


# speedup vs baseline: 1.6600x; 1.0031x over previous
import functools

import numpy as np
import jax
import jax.numpy as jnp
from jax.experimental import pallas as pl
from jax.experimental.pallas import tpu as pltpu
from jax.experimental.pallas import tpu_sc as plsc

_TINY = np.float32(np.finfo(np.float32).tiny)
_BLK = 8192
_SLC = 512
_SC_BLOCKS = 32
_SC_CHUNK = 8192

_K0 = np.uint32(0)
_K1 = np.uint32(42)
_KS2 = np.uint32(np.uint32(0x1BD11BDA) ^ _K0 ^ _K1)
_ROT0 = (13, 15, 26, 6)
_ROT1 = (17, 29, 16, 24)


def _rotl(x, d):
    return (x << np.uint32(d)) | (x >> np.uint32(32 - d))


def _threefry_xor(x1):
    def rounds(x0, x1, rots):
        for r in rots:
            x0 = x0 + x1
            x1 = x0 ^ _rotl(x1, r)
        return x0, x1

    x0 = x1
    x1 = x0 ^ _rotl(x1, _ROT0[0])
    x0, x1 = rounds(x0, x1, _ROT0[1:])
    x0 = x0 + _K1
    x1 = x1 + np.uint32(_KS2 + np.uint32(1))
    x0, x1 = rounds(x0, x1, _ROT1)
    x0 = x0 + _KS2
    x1 = x1 + np.uint32(_K0 + np.uint32(2))
    x0, x1 = rounds(x0, x1, _ROT0)
    x0 = x0 + _K0
    x1 = x1 + np.uint32(_K1 + np.uint32(3))
    x0, x1 = rounds(x0, x1, _ROT1)
    x0 = x0 + _K1
    x1 = x1 + np.uint32(_KS2 + np.uint32(4))
    x0, x1 = rounds(x0, x1, _ROT0)
    x0 = x0 + _KS2
    x1 = x1 + np.uint32(_K0 + np.uint32(5))
    return x0 ^ x1


def _bits_to_u(bits):
    fb = (bits >> np.uint32(9)) | np.uint32(0x3F800000)
    f = jax.lax.bitcast_convert_type(fb, jnp.float32) - np.float32(1.0)
    return jnp.maximum(_TINY, f * np.float32(np.float32(1.0) - _TINY) + _TINY)


def _sc_body(u_ref, buf, *, V, col0, n_cols):
    b = jax.lax.axis_index("s") * 2 + jax.lax.axis_index("c")
    base = b * V + col0 + 42
    lane = jax.lax.iota(jnp.int32, 16)
    nch = n_cols // _SC_CHUNK

    def chunk(ci, _):
        def vec(vi, _):
            off = vi * 64
            for t in range(4):
                x1 = (lane + (base + ci * _SC_CHUNK + off + t * 16)
                      ).astype(jnp.uint32)
                buf[pl.ds(off + t * 16, 16)] = _bits_to_u(_threefry_xor(x1))
            return 0

        jax.lax.fori_loop(0, _SC_CHUNK // 64, vec, 0)
        pltpu.sync_copy(buf, u_ref.at[b, pl.ds(ci * _SC_CHUNK, _SC_CHUNK)])
        return 0

    jax.lax.fori_loop(0, nch, chunk, 0)


def _sc_u_values(B, V, col0, n_cols):
    mesh = plsc.VectorSubcoreMesh(core_axis_name="c", subcore_axis_name="s")
    body = functools.partial(_sc_body, V=V, col0=col0, n_cols=n_cols)
    try:
        fn = pl.kernel(
            body, mesh=mesh,
            out_type=jax.ShapeDtypeStruct((B, n_cols), jnp.float32),
            scratch_types=[pltpu.VMEM((_SC_CHUNK,), jnp.float32)],
        )
    except TypeError:
        fn = pl.kernel(
            body, mesh=mesh,
            out_shape=jax.ShapeDtypeStruct((B, n_cols), jnp.float32),
            scratch_shapes=[pltpu.VMEM((_SC_CHUNK,), jnp.float32)],
        )
    return fn()


def _tc1_kernel(logits_ref, vmax_ref, vidx_ref, *, V, B):
    k = pl.program_id(0)

    @pl.when(k == 0)
    def _init():
        vmax_ref[...] = jnp.full((B, _SLC), -jnp.inf, jnp.float32)
        vidx_ref[...] = jnp.zeros((B, _SLC), jnp.int32)

    nslc = _BLK // _SLC
    base_col = k * _BLK
    rv = (jax.lax.broadcasted_iota(jnp.uint32, (B, _SLC), 0) * np.uint32(V)
          + jax.lax.broadcasted_iota(jnp.uint32, (B, _SLC), 1))

    def body(s, carry):
        vmax, vidx = carry
        col0 = s * _SLC
        x1 = rv + (base_col + col0 + 42).astype(jnp.uint32)
        u = _bits_to_u(_threefry_xor(x1))
        g = -jnp.log(-jnp.log(u))
        vals = logits_ref[:, pl.ds(col0, _SLC)] + g
        upd = vals > vmax
        sid = jnp.full((B, _SLC), 0, jnp.int32) + (k * nslc + s)
        return jnp.where(upd, vals, vmax), jnp.where(upd, sid, vidx)

    vmax, vidx = jax.lax.fori_loop(
        0, nslc, body, (vmax_ref[...], vidx_ref[...]))
    vmax_ref[...] = vmax
    vidx_ref[...] = vidx


def _tc2_kernel(logits_ref, u_ref, pv_ref, pi_ref, out_ref, vmax_ref, vidx_ref,
                *, nblk2, k_off, V, B):
    k = pl.program_id(0)

    @pl.when(k == 0)
    def _init():
        vmax_ref[...] = pv_ref[...]
        vidx_ref[...] = pi_ref[...]

    nslc = _BLK // _SLC
    kg = k + k_off
    base_col = kg * _BLK
    coli = jax.lax.broadcasted_iota(jnp.int32, (B, _SLC), 1)

    def body(s2, carry):
        vmax, vidx = carry
        vv = []
        for t in range(2):
            s = s2 * 2 + t
            col0 = s * _SLC
            g = -jnp.log(-jnp.log(u_ref[:, pl.ds(col0, _SLC)]))
            vals = logits_ref[:, pl.ds(col0, _SLC)] + g
            vals = jnp.where(coli < V - (base_col + col0), vals, -jnp.inf)
            vv.append((vals, s))
        for vals, s in vv:
            upd = vals > vmax
            sid = jnp.full((B, _SLC), 0, jnp.int32) + (kg * nslc + s)
            vmax = jnp.where(upd, vals, vmax)
            vidx = jnp.where(upd, sid, vidx)
        return vmax, vidx

    vmax, vidx = jax.lax.fori_loop(
        0, nslc // 2, body, (vmax_ref[...], vidx_ref[...]))
    vmax_ref[...] = vmax
    vidx_ref[...] = vidx

    @pl.when(k == nblk2 - 1)
    def _done():
        vm = vmax_ref[...]
        m = jnp.max(vm, axis=1, keepdims=True)
        gidx = vidx_ref[...] * _SLC + coli
        out_ref[...] = jnp.min(
            jnp.where(vm == m, gidx, np.int32(2**30)), axis=1, keepdims=True)


def kernel(logits):
    B, V = logits.shape
    nblk = (V + _BLK - 1) // _BLK
    nblk2 = min(_SC_BLOCKS, nblk - 1)
    nblk1 = nblk - nblk2

    u_tail = _sc_u_values(B, V, nblk1 * _BLK, nblk2 * _BLK)

    pv, pi = pl.pallas_call(
        functools.partial(_tc1_kernel, V=V, B=B),
        grid=(nblk1,),
        in_specs=[pl.BlockSpec((B, _BLK), lambda k: (0, k))],
        out_specs=[pl.BlockSpec((B, _SLC), lambda k: (0, 0)),
                   pl.BlockSpec((B, _SLC), lambda k: (0, 0))],
        out_shape=[jax.ShapeDtypeStruct((B, _SLC), jnp.float32),
                   jax.ShapeDtypeStruct((B, _SLC), jnp.int32)],
    )(logits)

    return pl.pallas_call(
        functools.partial(_tc2_kernel, nblk2=nblk2, k_off=nblk1, V=V, B=B),
        grid=(nblk2,),
        in_specs=[
            pl.BlockSpec((B, _BLK), lambda k, o=nblk1: (0, k + o)),
            pl.BlockSpec((B, _BLK), lambda k: (0, k)),
            pl.BlockSpec((B, _SLC), lambda k: (0, 0)),
            pl.BlockSpec((B, _SLC), lambda k: (0, 0)),
        ],
        out_specs=pl.BlockSpec((B, 1), lambda k: (0, 0)),
        out_shape=jax.ShapeDtypeStruct((B, 1), jnp.int32),
        scratch_shapes=[pltpu.VMEM((B, _SLC), jnp.float32),
                        pltpu.VMEM((B, _SLC), jnp.int32)],
    )(logits, u_tail, pv, pi)
